# baseline (speedup 1.0000x reference)
import jax
import jax.numpy as jnp
from jax import lax
from jax.experimental import pallas as pl
from jax.experimental.pallas import tpu as pltpu
from jax.experimental.pallas import tpu_sc as plsc

N_NODES = 50000
N_EDGES = 800000
NC, NS, L = 2, 16, 16
NW = NC * NS
D = 16
H = 64
EPR = 128 // D

_SC_PARAMS = pltpu.CompilerParams(use_tc_tiling_on_sc=False,
                                  needs_layout_passes=False)

G = 128
N_CHUNKS = (2 * N_EDGES) // G
KDEPTH = 8
N_GROUPS = -(-N_CHUNKS // (NW * KDEPTH))
FROWS = 2 * N_EDGES * D // 128


def _gather_body(table_hbm, idx_hbm, out_hbm, idx_v, rows_v, isem, gsem):
    c = lax.axis_index("c")
    s = lax.axis_index("s")
    w = s * NC + c

    def group(jg, _):
        base = w + NW * KDEPTH * jg
        for b in range(KDEPTH):
            chunk = base + NW * b
            @pl.when(chunk < N_CHUNKS)
            def _():
                pltpu.async_copy(idx_hbm.at[pl.ds(chunk * G, G)], idx_v.at[b], isem)
        for b in range(KDEPTH):
            chunk = base + NW * b
            @pl.when(chunk < N_CHUNKS)
            def _():
                pltpu.make_async_copy(idx_hbm.at[pl.ds(chunk * G, G)], idx_v.at[b], isem).wait()
                pltpu.async_copy(table_hbm.at[idx_v.at[b]], rows_v.at[b], gsem)
        for b in range(KDEPTH):
            chunk = base + NW * b
            @pl.when(chunk < N_CHUNKS)
            def _():
                pltpu.make_async_copy(table_hbm.at[idx_v.at[b]], rows_v.at[b], gsem).wait()
                pltpu.sync_copy(rows_v.at[b], out_hbm.at[chunk])
        return 0

    lax.fori_loop(0, N_GROUPS, group, 0)


def _k1_gather(table, idx_flat):
    mesh = plsc.VectorSubcoreMesh(core_axis_name="c", subcore_axis_name="s")
    return pl.kernel(
        _gather_body,
        out_type=jax.ShapeDtypeStruct((N_CHUNKS, G, D), jnp.float32),
        mesh=mesh,
        scratch_types=[
            pltpu.VMEM((KDEPTH, G), jnp.int32),
            pltpu.VMEM((KDEPTH, G, D), jnp.float32),
            pltpu.SemaphoreType.DMA,
            pltpu.SemaphoreType.DMA,
        ],
        compiler_params=_SC_PARAMS,
    )(table, idx_flat)


BR = 400
NB = (N_EDGES * D) // (128 * BR)
MROWS = N_EDGES * H // 512


def _celu(v):
    return jnp.where(v > 0, v, jnp.exp(v) - 1.0)


def _edge_mlp_body(rs_ref, rd_ref, w1b_ref, b1_ref, w2_ref, b2_ref, out_ref):
    col = lax.broadcasted_iota(jnp.int32, (BR, 128), 1) % D
    sub = (col >= 3) & (col < 6)
    feat = rs_ref[...] - jnp.where(sub, rd_ref[...], 0.0)
    h = jnp.dot(feat, w1b_ref[...], preferred_element_type=jnp.float32) + b1_ref[...]
    h = _celu(h)
    w2 = w2_ref[...]
    b2 = b2_ref[...]
    parts = []
    for j in range(EPR):
        m = jnp.dot(h[:, j * H:(j + 1) * H], w2,
                    preferred_element_type=jnp.float32) + b2
        parts.append(jnp.maximum(m, 0.0))
    for p in range(EPR // 2):
        out_ref[p] = jnp.concatenate([parts[2 * p], parts[2 * p + 1]], axis=1)


def _k2_edge_mlp(feat, w1b, b1t, f_w2, f_b2):
    half = FROWS // 2
    return pl.pallas_call(
        _edge_mlp_body,
        grid=(NB,),
        in_specs=[
            pl.BlockSpec((BR, 128), lambda i: (i, 0)),
            pl.BlockSpec((BR, 128), lambda i: (i + half // BR, 0)),
            pl.BlockSpec((128, 8 * H), lambda i: (0, 0)),
            pl.BlockSpec((1, 8 * H), lambda i: (0, 0)),
            pl.BlockSpec((H, H), lambda i: (0, 0)),
            pl.BlockSpec((1, H), lambda i: (0, 0)),
        ],
        out_specs=pl.BlockSpec((EPR // 2, BR, 128), lambda i: (0, i, 0)),
        out_shape=jax.ShapeDtypeStruct((EPR // 2, MROWS, 128), jnp.float32),
    )(feat, feat, w1b, b1t, f_w2, f_b2.reshape(1, H))


RPW = 1568
NP = NW * RPW
TRASH = RPW
CH = 1600
NCH = N_EDGES // CH
VB = CH // L
BATCH = 128
PCAP = 2048
RMASK = PCAP - 1
AROWS = RPW + 8
GROWS = N_EDGES // 2


def _scatter_body(dst_hbm, msg_hbm, out_hbm, dbuf, pend_i, pend_d,
                  rows_v, aggr_v, csem, gsem):
    c = lax.axis_index("c")
    s = lax.axis_index("s")
    w = s * NC + c
    lo = w * RPW
    zero16 = jnp.zeros((L,), jnp.float32)
    iota16 = lax.iota(jnp.int32, L)

    def zi(i, _):
        aggr_v[pl.ds(i * L, L)] = zero16
        return 0
    lax.fori_loop(0, AROWS * H // L, zi, 0)

    def chunk_start(ci, half):
        pltpu.async_copy(dst_hbm.at[pl.ds(ci * CH, CH)],
                         dbuf.at[pl.ds(half * CH, CH)], csem)

    def chunk_wait(ci, half):
        pltpu.make_async_copy(dst_hbm.at[pl.ds(ci * CH, CH)],
                              dbuf.at[pl.ds(half * CH, CH)], csem).wait()

    rconst = 2 * ((lax.shift_right_logical(iota16, 1) & 3) * MROWS
                  + lax.shift_right_logical(iota16, 3)) + (iota16 & 1)

    def fire(rp, slot):
        base = pl.multiple_of(rp & RMASK, BATCH)
        pltpu.async_copy(
            msg_hbm.at[pend_i.at[pl.ds(base, BATCH)]],
            rows_v.at[slot], gsem)

    def wait_g(rp, slot):
        base = pl.multiple_of(rp & RMASK, BATCH)
        pltpu.make_async_copy(
            msg_hbm.at[pend_i.at[pl.ds(base, BATCH)]],
            rows_v.at[slot], gsem).wait()

    def rmw_batch(rp, slot):
        base = pl.multiple_of(rp & RMASK, BATCH)
        UE = 4

        def rmw(i4, _):
            vs = []
            for u in range(UE):
                i = UE * i4 + u
                dvec = pend_d[pl.ds(pl.multiple_of(base + (i & -L), L), L)]
                lane = i & (L - 1)
                vs.append(lax.reduce_sum(
                    jnp.where(iota16 == lane, dvec, 0), axes=(0,)))
            for u in range(UE):
                i = UE * i4 + u
                rb = vs[u] * H
                a = [aggr_v[pl.ds(rb + k * L, L)] for k in range(H // L)]
                r = [rows_v[slot, i, pl.ds(k * L, L)] for k in range(H // L)]
                for k in range(H // L):
                    aggr_v[pl.ds(rb + k * L, L)] = jnp.maximum(a[k], r[k])
            return 0
        lax.fori_loop(0, BATCH // UE, rmw, 0)

    chunk_start(0, 0)

    UN = 5

    def chunk_body(ci, carry):
        wp, rp = carry
        @pl.when(ci + 1 < NCH)
        def _():
            chunk_start(ci + 1, (ci + 1) % 2)
        chunk_wait(ci, ci % 2)
        dbase = (ci % 2) * CH

        f1 = wp - rp >= BATCH
        f2 = wp - rp >= 2 * BATCH
        @pl.when(f1)
        def _():
            fire(rp, 0)
        @pl.when(f2)
        def _():
            fire(rp + BATCH, 1)

        voff0 = jnp.full((L,), wp - 1, jnp.int32)
        ev0 = jnp.full((L,), 400 * ci, jnp.int32) + rconst

        def scan_vreg(vi, carry):
            voff, ev = carry
            for u in range(UN):
                v = vi * UN + u
                dvec = dbuf[pl.ds(dbase + v * L, L)]
                dloc = dvec - lo
                mask = plsc.bitcast(dloc, jnp.uint32) < jnp.uint32(RPW)
                pc = plsc.cumsum(jnp.where(mask, 1, 0))
                pos = (voff + pc) & RMASK
                plsc.store_scatter(pend_i, [pos], ev, mask=mask)
                plsc.store_scatter(pend_d, [pos], dloc, mask=mask)
                voff = voff + plsc.all_reduce_population_count(mask)
                ev = ev + 4
            return voff, ev

        voff, _ = lax.fori_loop(0, VB // UN, scan_vreg, (voff0, ev0))
        wp = lax.reduce_max(voff, axes=(0,)) + 1

        @pl.when(f1)
        def _():
            wait_g(rp, 0)
            rmw_batch(rp, 0)
        @pl.when(f2)
        def _():
            wait_g(rp + BATCH, 1)
            rmw_batch(rp + BATCH, 1)
        rp = rp + jnp.where(f2, 2 * BATCH, jnp.where(f1, BATCH, 0))

        wp, rp = lax.while_loop(lambda c: c[0] - c[1] >= 320, _drain,
                                (wp, rp))
        return wp, rp

    def _drain(carry):
        wp, rp = carry
        two = wp - rp >= 2 * BATCH
        fire(rp, 0)
        @pl.when(two)
        def _():
            fire(rp + BATCH, 1)
        wait_g(rp, 0)
        rmw_batch(rp, 0)
        @pl.when(two)
        def _():
            wait_g(rp + BATCH, 1)
            rmw_batch(rp + BATCH, 1)
        return wp, rp + jnp.where(two, 2 * BATCH, BATCH)

    wp, rp = lax.fori_loop(0, NCH, chunk_body, (0, 0))
    wp, rp = lax.while_loop(lambda c: c[0] - c[1] >= BATCH, _drain, (wp, rp))

    for j in range(BATCH // L):
        pos = (wp + j * L + iota16) & RMASK
        full = jnp.full((L,), True, jnp.bool_)
        plsc.store_scatter(pend_i, [pos], w * 64 + j * L + iota16, mask=full)
        plsc.store_scatter(pend_d, [pos], jnp.full((L,), TRASH, jnp.int32),
                           mask=full)
    fire(rp, 0)
    wait_g(rp, 0)
    rmw_batch(rp, 0)

    pltpu.sync_copy(aggr_v.at[pl.ds(0, RPW * H)],
                    out_hbm.at[pl.ds(w * RPW * H, RPW * H)])


def _k3_segment_max(dst, msg2):
    mesh = plsc.VectorSubcoreMesh(core_axis_name="c", subcore_axis_name="s")
    return pl.kernel(
        _scatter_body,
        out_type=jax.ShapeDtypeStruct((NP * H,), jnp.float32),
        mesh=mesh,
        scratch_types=[
            pltpu.VMEM((2 * CH,), jnp.int32),
            pltpu.VMEM((PCAP,), jnp.int32),
            pltpu.VMEM((PCAP,), jnp.int32),
            pltpu.VMEM((2, BATCH, H), jnp.float32),
            pltpu.VMEM((AROWS * H,), jnp.float32),
            pltpu.SemaphoreType.DMA,
            pltpu.SemaphoreType.DMA,
        ],
        compiler_params=_SC_PARAMS,
    )(dst, msg2)


BN = 784


def _update_body(a_ref, xp_ref, g1a_ref, g1x_ref, gb1_ref, g2_ref, gb2_ref, out_ref):
    a = a_ref[...]
    xp = xp_ref[...]
    g1a = g1a_ref[...]
    g1x = g1x_ref[...]
    gb1 = gb1_ref[...]
    g2 = g2_ref[...]
    gb2 = gb2_ref[...]
    outs = []
    for j in range(2):
        h = (jnp.dot(a[:, j * H:(j + 1) * H], g1a, preferred_element_type=jnp.float32)
             + jnp.dot(xp[:, j * 3:(j + 1) * 3], g1x, preferred_element_type=jnp.float32)
             + gb1)
        outs.append(_celu(jnp.dot(_celu(h), g2, preferred_element_type=jnp.float32) + gb2))
    out_ref[...] = jnp.concatenate(outs, axis=1)


def _k4_update(aggr2, xp, g1a, g1x, g_b1, g_w2, g_b2):
    n2 = NP // 2
    return pl.pallas_call(
        _update_body,
        grid=(n2 // BN,),
        in_specs=[
            pl.BlockSpec((BN, 128), lambda i: (i, 0)),
            pl.BlockSpec((BN, 6), lambda i: (i, 0)),
            pl.BlockSpec((H, H), lambda i: (0, 0)),
            pl.BlockSpec((3, H), lambda i: (0, 0)),
            pl.BlockSpec((1, H), lambda i: (0, 0)),
            pl.BlockSpec((H, H), lambda i: (0, 0)),
            pl.BlockSpec((1, H), lambda i: (0, 0)),
        ],
        out_specs=pl.BlockSpec((BN, 128), lambda i: (i, 0)),
        out_shape=jax.ShapeDtypeStruct((n2, 128), jnp.float32),
    )(aggr2, xp, g1a, g1x, g_b1.reshape(1, H), g_w2, g_b2.reshape(1, H))


def kernel(x, pos, edge_index, f_w1, f_b1, f_w2, f_b2,
           g_w1, g_b1, g_w2, g_b2, h_w1, h_b1, h_w2, h_b2):
    table = jnp.concatenate(
        [x, pos, jnp.zeros((N_NODES, D - 6), jnp.float32)], axis=1)
    idx_flat = edge_index.reshape(2 * N_EDGES)
    feat = _k1_gather(table, idx_flat).reshape(FROWS, 128)

    w1p = jnp.concatenate([f_w1, jnp.zeros((D - 6, H), jnp.float32)], axis=0)
    w1b = jnp.zeros((EPR, D, EPR, H), jnp.float32)
    w1b = w1b.at[jnp.arange(EPR), :, jnp.arange(EPR), :].set(w1p)
    w1b = w1b.reshape(128, EPR * H)
    b1t = jnp.tile(f_b1, EPR).reshape(1, EPR * H)
    msg = _k2_edge_mlp(feat, w1b, b1t, f_w2, f_b2)

    msg2 = msg.reshape(N_EDGES, H)
    aggr2 = _k3_segment_max(edge_index[1], msg2).reshape(NP // 2, 128)

    xp = jnp.concatenate(
        [x.reshape(N_NODES // 2, 6),
         jnp.zeros(((NP - N_NODES) // 2, 6), jnp.float32)], axis=0)
    out2 = _k4_update(aggr2, xp, g_w1[:H], g_w1[H:], g_b1, g_w2, g_b2)
    out = out2.reshape(NP, H)[:N_NODES]
    return (out, pos, edge_index)


# speedup vs baseline: 1.0025x; 1.0025x over previous
import jax
import jax.numpy as jnp
from jax import lax
from jax.experimental import pallas as pl
from jax.experimental.pallas import tpu as pltpu
from jax.experimental.pallas import tpu_sc as plsc

N_NODES = 50000
N_EDGES = 800000
NC, NS, L = 2, 16, 16
NW = NC * NS
D = 16
H = 64
EPR = 128 // D

_SC_PARAMS = pltpu.CompilerParams(use_tc_tiling_on_sc=False,
                                  needs_layout_passes=False)

G = 128
N_CHUNKS = (2 * N_EDGES) // G
KDEPTH = 8
N_GROUPS = -(-N_CHUNKS // (NW * KDEPTH))
FROWS = 2 * N_EDGES * D // 128


def _gather_body(table_hbm, idx_hbm, out_hbm, idx_v, rows_v, isem, gsem):
    c = lax.axis_index("c")
    s = lax.axis_index("s")
    w = s * NC + c

    def group(jg, _):
        base = w + NW * KDEPTH * jg
        for b in range(KDEPTH):
            chunk = base + NW * b
            @pl.when(chunk < N_CHUNKS)
            def _():
                pltpu.async_copy(idx_hbm.at[pl.ds(chunk * G, G)], idx_v.at[b], isem)
        for b in range(KDEPTH):
            chunk = base + NW * b
            @pl.when(chunk < N_CHUNKS)
            def _():
                pltpu.make_async_copy(idx_hbm.at[pl.ds(chunk * G, G)], idx_v.at[b], isem).wait()
                pltpu.async_copy(table_hbm.at[idx_v.at[b]], rows_v.at[b], gsem)
        for b in range(KDEPTH):
            chunk = base + NW * b
            @pl.when(chunk < N_CHUNKS)
            def _():
                pltpu.make_async_copy(table_hbm.at[idx_v.at[b]], rows_v.at[b], gsem).wait()
                pltpu.sync_copy(rows_v.at[b], out_hbm.at[chunk])
        return 0

    lax.fori_loop(0, N_GROUPS, group, 0)


def _k1_gather(table, idx_flat):
    mesh = plsc.VectorSubcoreMesh(core_axis_name="c", subcore_axis_name="s")
    return pl.kernel(
        _gather_body,
        out_type=jax.ShapeDtypeStruct((N_CHUNKS, G, D), jnp.float32),
        mesh=mesh,
        scratch_types=[
            pltpu.VMEM((KDEPTH, G), jnp.int32),
            pltpu.VMEM((KDEPTH, G, D), jnp.float32),
            pltpu.SemaphoreType.DMA,
            pltpu.SemaphoreType.DMA,
        ],
        compiler_params=_SC_PARAMS,
    )(table, idx_flat)


BR = 400
NB = (N_EDGES * D) // (128 * BR)
MROWS = N_EDGES * H // 512


def _celu(v):
    return jnp.where(v > 0, v, jnp.exp(v) - 1.0)


def _edge_mlp_body(rs_ref, rd_ref, w1b_ref, b1_ref, w2_ref, b2_ref, out_ref):
    col = lax.broadcasted_iota(jnp.int32, (BR, 128), 1) % D
    sub = (col >= 3) & (col < 6)
    feat = rs_ref[...] - jnp.where(sub, rd_ref[...], 0.0)
    h = jnp.dot(feat, w1b_ref[...], preferred_element_type=jnp.float32) + b1_ref[...]
    h = _celu(h)
    w2 = w2_ref[...]
    b2 = b2_ref[...]
    parts = []
    for j in range(EPR):
        m = jnp.dot(h[:, j * H:(j + 1) * H], w2,
                    preferred_element_type=jnp.float32) + b2
        parts.append(jnp.maximum(m, 0.0))
    for p in range(EPR // 2):
        out_ref[p] = jnp.concatenate([parts[2 * p], parts[2 * p + 1]], axis=1)


def _k2_edge_mlp(feat, w1b, b1t, f_w2, f_b2):
    half = FROWS // 2
    return pl.pallas_call(
        _edge_mlp_body,
        grid=(NB,),
        in_specs=[
            pl.BlockSpec((BR, 128), lambda i: (i, 0)),
            pl.BlockSpec((BR, 128), lambda i: (i + half // BR, 0)),
            pl.BlockSpec((128, 8 * H), lambda i: (0, 0)),
            pl.BlockSpec((1, 8 * H), lambda i: (0, 0)),
            pl.BlockSpec((H, H), lambda i: (0, 0)),
            pl.BlockSpec((1, H), lambda i: (0, 0)),
        ],
        out_specs=pl.BlockSpec((EPR // 2, BR, 128), lambda i: (0, i, 0)),
        out_shape=jax.ShapeDtypeStruct((EPR // 2, MROWS, 128), jnp.float32),
    )(feat, feat, w1b, b1t, f_w2, f_b2.reshape(1, H))


RPW = 1568
NP = NW * RPW
TRASH = RPW
CH = 1600
NCH = N_EDGES // CH
VB = CH // L
BATCH = 128
PCAP = 2048
RMASK = PCAP - 1
AROWS = RPW + 8
GROWS = N_EDGES // 2


def _scatter_body(dst_hbm, msg_hbm, out_hbm, dbuf, pend_i, pend_d,
                  rows_v, aggr_v, csem, gsem):
    c = lax.axis_index("c")
    s = lax.axis_index("s")
    w = s * NC + c
    lo = w * RPW
    zero16 = jnp.zeros((L,), jnp.float32)
    iota16 = lax.iota(jnp.int32, L)

    def zi(i, _):
        aggr_v[pl.ds(i * L, L)] = zero16
        return 0
    lax.fori_loop(0, AROWS * H // L, zi, 0)

    def chunk_start(ci, half):
        pltpu.async_copy(dst_hbm.at[pl.ds(ci * CH, CH)],
                         dbuf.at[pl.ds(half * CH, CH)], csem)

    def chunk_wait(ci, half):
        pltpu.make_async_copy(dst_hbm.at[pl.ds(ci * CH, CH)],
                              dbuf.at[pl.ds(half * CH, CH)], csem).wait()

    rconst = 2 * ((lax.shift_right_logical(iota16, 1) & 3) * MROWS
                  + lax.shift_right_logical(iota16, 3)) + (iota16 & 1)

    def fire(rp, slot):
        base = pl.multiple_of(rp & RMASK, BATCH)
        pltpu.async_copy(
            msg_hbm.at[pend_i.at[pl.ds(base, BATCH)]],
            rows_v.at[slot], gsem)

    def wait_g(rp, slot):
        base = pl.multiple_of(rp & RMASK, BATCH)
        pltpu.make_async_copy(
            msg_hbm.at[pend_i.at[pl.ds(base, BATCH)]],
            rows_v.at[slot], gsem).wait()

    def rmw_batch(rp, slot):
        base = pl.multiple_of(rp & RMASK, BATCH)
        UE = 4

        def rmw(i4, _):
            vs = []
            for u in range(UE):
                i = UE * i4 + u
                dvec = pend_d[pl.ds(pl.multiple_of(base + (i & -L), L), L)]
                lane = i & (L - 1)
                vs.append(lax.reduce_sum(
                    jnp.where(iota16 == lane, dvec, 0), axes=(0,)))
            for u in range(UE):
                i = UE * i4 + u
                rb = vs[u] * H
                a = [aggr_v[pl.ds(rb + k * L, L)] for k in range(H // L)]
                r = [rows_v[slot, i, pl.ds(k * L, L)] for k in range(H // L)]
                for k in range(H // L):
                    aggr_v[pl.ds(rb + k * L, L)] = jnp.maximum(a[k], r[k])
            return 0
        lax.fori_loop(0, BATCH // UE, rmw, 0)

    chunk_start(0, 0)

    UN = 10

    def chunk_body(ci, carry):
        wp, rp = carry
        @pl.when(ci + 1 < NCH)
        def _():
            chunk_start(ci + 1, (ci + 1) % 2)
        chunk_wait(ci, ci % 2)
        dbase = (ci % 2) * CH

        f1 = wp - rp >= BATCH
        f2 = wp - rp >= 2 * BATCH
        @pl.when(f1)
        def _():
            fire(rp, 0)
        @pl.when(f2)
        def _():
            fire(rp + BATCH, 1)

        voff0 = jnp.full((L,), wp - 1, jnp.int32)
        ev0 = jnp.full((L,), 400 * ci, jnp.int32) + rconst

        def scan_vreg(vi, carry):
            voff, ev = carry
            for u in range(UN):
                v = vi * UN + u
                dvec = dbuf[pl.ds(dbase + v * L, L)]
                dloc = dvec - lo
                mask = plsc.bitcast(dloc, jnp.uint32) < jnp.uint32(RPW)
                pc = plsc.cumsum(jnp.where(mask, 1, 0))
                pos = (voff + pc) & RMASK
                plsc.store_scatter(pend_i, [pos], ev, mask=mask)
                plsc.store_scatter(pend_d, [pos], dloc, mask=mask)
                voff = voff + plsc.all_reduce_population_count(mask)
                ev = ev + 4
            return voff, ev

        voff, _ = lax.fori_loop(0, VB // UN, scan_vreg, (voff0, ev0))
        wp = lax.reduce_max(voff, axes=(0,)) + 1

        @pl.when(f1)
        def _():
            wait_g(rp, 0)
            rmw_batch(rp, 0)
        @pl.when(f2)
        def _():
            wait_g(rp + BATCH, 1)
            rmw_batch(rp + BATCH, 1)
        rp = rp + jnp.where(f2, 2 * BATCH, jnp.where(f1, BATCH, 0))

        wp, rp = lax.while_loop(lambda c: c[0] - c[1] >= 320, _drain,
                                (wp, rp))
        return wp, rp

    def _drain(carry):
        wp, rp = carry
        two = wp - rp >= 2 * BATCH
        fire(rp, 0)
        @pl.when(two)
        def _():
            fire(rp + BATCH, 1)
        wait_g(rp, 0)
        rmw_batch(rp, 0)
        @pl.when(two)
        def _():
            wait_g(rp + BATCH, 1)
            rmw_batch(rp + BATCH, 1)
        return wp, rp + jnp.where(two, 2 * BATCH, BATCH)

    wp, rp = lax.fori_loop(0, NCH, chunk_body, (0, 0))
    wp, rp = lax.while_loop(lambda c: c[0] - c[1] >= BATCH, _drain, (wp, rp))

    for j in range(BATCH // L):
        pos = (wp + j * L + iota16) & RMASK
        full = jnp.full((L,), True, jnp.bool_)
        plsc.store_scatter(pend_i, [pos], w * 64 + j * L + iota16, mask=full)
        plsc.store_scatter(pend_d, [pos], jnp.full((L,), TRASH, jnp.int32),
                           mask=full)
    fire(rp, 0)
    wait_g(rp, 0)
    rmw_batch(rp, 0)

    pltpu.sync_copy(aggr_v.at[pl.ds(0, RPW * H)],
                    out_hbm.at[pl.ds(w * RPW * H, RPW * H)])


def _k3_segment_max(dst, msg2):
    mesh = plsc.VectorSubcoreMesh(core_axis_name="c", subcore_axis_name="s")
    return pl.kernel(
        _scatter_body,
        out_type=jax.ShapeDtypeStruct((NP * H,), jnp.float32),
        mesh=mesh,
        scratch_types=[
            pltpu.VMEM((2 * CH,), jnp.int32),
            pltpu.VMEM((PCAP,), jnp.int32),
            pltpu.VMEM((PCAP,), jnp.int32),
            pltpu.VMEM((2, BATCH, H), jnp.float32),
            pltpu.VMEM((AROWS * H,), jnp.float32),
            pltpu.SemaphoreType.DMA,
            pltpu.SemaphoreType.DMA,
        ],
        compiler_params=_SC_PARAMS,
    )(dst, msg2)


BN = 784


def _update_body(a_ref, xp_ref, g1a_ref, g1x_ref, gb1_ref, g2_ref, gb2_ref, out_ref):
    a = a_ref[...]
    xp = xp_ref[...]
    g1a = g1a_ref[...]
    g1x = g1x_ref[...]
    gb1 = gb1_ref[...]
    g2 = g2_ref[...]
    gb2 = gb2_ref[...]
    outs = []
    for j in range(2):
        h = (jnp.dot(a[:, j * H:(j + 1) * H], g1a, preferred_element_type=jnp.float32)
             + jnp.dot(xp[:, j * 3:(j + 1) * 3], g1x, preferred_element_type=jnp.float32)
             + gb1)
        outs.append(_celu(jnp.dot(_celu(h), g2, preferred_element_type=jnp.float32) + gb2))
    out_ref[...] = jnp.concatenate(outs, axis=1)


def _k4_update(aggr2, xp, g1a, g1x, g_b1, g_w2, g_b2):
    n2 = NP // 2
    return pl.pallas_call(
        _update_body,
        grid=(n2 // BN,),
        in_specs=[
            pl.BlockSpec((BN, 128), lambda i: (i, 0)),
            pl.BlockSpec((BN, 6), lambda i: (i, 0)),
            pl.BlockSpec((H, H), lambda i: (0, 0)),
            pl.BlockSpec((3, H), lambda i: (0, 0)),
            pl.BlockSpec((1, H), lambda i: (0, 0)),
            pl.BlockSpec((H, H), lambda i: (0, 0)),
            pl.BlockSpec((1, H), lambda i: (0, 0)),
        ],
        out_specs=pl.BlockSpec((BN, 128), lambda i: (i, 0)),
        out_shape=jax.ShapeDtypeStruct((n2, 128), jnp.float32),
    )(aggr2, xp, g1a, g1x, g_b1.reshape(1, H), g_w2, g_b2.reshape(1, H))


def kernel(x, pos, edge_index, f_w1, f_b1, f_w2, f_b2,
           g_w1, g_b1, g_w2, g_b2, h_w1, h_b1, h_w2, h_b2):
    table = jnp.concatenate(
        [x, pos, jnp.zeros((N_NODES, D - 6), jnp.float32)], axis=1)
    idx_flat = edge_index.reshape(2 * N_EDGES)
    feat = _k1_gather(table, idx_flat).reshape(FROWS, 128)

    w1p = jnp.concatenate([f_w1, jnp.zeros((D - 6, H), jnp.float32)], axis=0)
    w1b = jnp.zeros((EPR, D, EPR, H), jnp.float32)
    w1b = w1b.at[jnp.arange(EPR), :, jnp.arange(EPR), :].set(w1p)
    w1b = w1b.reshape(128, EPR * H)
    b1t = jnp.tile(f_b1, EPR).reshape(1, EPR * H)
    msg = _k2_edge_mlp(feat, w1b, b1t, f_w2, f_b2)

    msg2 = msg.reshape(N_EDGES, H)
    aggr2 = _k3_segment_max(edge_index[1], msg2).reshape(NP // 2, 128)

    xp = jnp.concatenate(
        [x.reshape(N_NODES // 2, 6),
         jnp.zeros(((NP - N_NODES) // 2, 6), jnp.float32)], axis=0)
    out2 = _k4_update(aggr2, xp, g_w1[:H], g_w1[H:], g_b1, g_w2, g_b2)
    out = out2.reshape(NP, H)[:N_NODES]
    return (out, pos, edge_index)
